# baseline (speedup 1.0000x reference)
import functools
import math

import jax
import jax.numpy as jnp
from jax import lax
from jax.experimental import pallas as pl
from jax.experimental.pallas import tpu as pltpu
from jax.experimental.pallas import tpu_sc as plsc

D_MODEL = 64
SCALE = math.sqrt(D_MODEL)

_NC = 2
_NS = 16
_NW = _NC * _NS


@functools.partial(jax.jit, static_argnames=("total", "chunk"))
def _sc_embed(table, idx, *, total, chunk):
    b_per_w = total // _NW
    n_chunks = b_per_w // chunk
    mesh = plsc.VectorSubcoreMesh(core_axis_name="c", subcore_axis_name="s")

    @functools.partial(
        pl.kernel,
        mesh=mesh,
        out_type=jax.ShapeDtypeStruct((total, D_MODEL), jnp.float32),
        scratch_types=[
            pltpu.VMEM((b_per_w,), jnp.int32),
            pltpu.VMEM((chunk, D_MODEL), jnp.float32),
            pltpu.SemaphoreType.DMA,
        ],
        compiler_params=pltpu.CompilerParams(use_tc_tiling_on_sc=False),
    )
    def k(table_hbm, idx_hbm, out_hbm, idx_v, rows_v, sem):
        wid = lax.axis_index("s") * _NC + lax.axis_index("c")
        base = wid * b_per_w
        pltpu.sync_copy(idx_hbm.at[pl.ds(base, b_per_w)], idx_v)

        def chunk_body(g, carry):
            pltpu.async_copy(
                table_hbm.at[idx_v.at[pl.ds(g * chunk, chunk)]], rows_v, sem
            ).wait()

            def row_body(i, c2):
                for j in range(D_MODEL // 16):
                    sl = pl.ds(j * 16, 16)
                    rows_v[i, sl] = rows_v[i, sl] * SCALE
                return c2

            lax.fori_loop(0, chunk, row_body, 0, unroll=4)
            pltpu.sync_copy(rows_v, out_hbm.at[pl.ds(base + g * chunk, chunk)])
            return carry

        lax.fori_loop(0, n_chunks, chunk_body, 0)

    return k(table, idx)


def kernel(x, table):
    total = x.shape[0] * x.shape[1]
    idx = x.reshape(total).astype(jnp.int32)
    out = _sc_embed(table, idx, total=total, chunk=512)
    return out.reshape(x.shape + (D_MODEL,))


# speedup vs baseline: 1.0622x; 1.0622x over previous
import functools
import math

import jax
import jax.numpy as jnp
from jax import lax
from jax.experimental import pallas as pl
from jax.experimental.pallas import tpu as pltpu
from jax.experimental.pallas import tpu_sc as plsc

D_MODEL = 64
SCALE = math.sqrt(D_MODEL)

_NC = 2
_NS = 16
_NW = _NC * _NS
_NBUF = 3


@functools.partial(jax.jit, static_argnames=("total", "chunk"))
def _sc_embed(table, idx, *, total, chunk):
    b_per_w = total // _NW
    n_chunks = b_per_w // chunk
    mesh = plsc.VectorSubcoreMesh(core_axis_name="c", subcore_axis_name="s")

    @functools.partial(
        pl.kernel,
        mesh=mesh,
        out_type=jax.ShapeDtypeStruct((total, D_MODEL), jnp.float32),
        scratch_types=[
            pltpu.VMEM((b_per_w,), jnp.int32),
        ]
        + [pltpu.VMEM((chunk, D_MODEL), jnp.float32) for _ in range(_NBUF)]
        + [pltpu.SemaphoreType.DMA for _ in range(2 * _NBUF)],
        compiler_params=pltpu.CompilerParams(use_tc_tiling_on_sc=False),
    )
    def k(table_hbm, idx_hbm, out_hbm, idx_v, *bufs_and_sems):
        bufs = bufs_and_sems[:_NBUF]
        sem_g = bufs_and_sems[_NBUF:2 * _NBUF]
        sem_s = bufs_and_sems[2 * _NBUF:]

        wid = lax.axis_index("s") * _NC + lax.axis_index("c")
        base = wid * b_per_w
        pltpu.sync_copy(idx_hbm.at[pl.ds(base, b_per_w)], idx_v)

        def start_gather(g):
            b = g % _NBUF
            return pltpu.async_copy(
                table_hbm.at[idx_v.at[pl.ds(g * chunk, chunk)]],
                bufs[b], sem_g[b])

        def start_scatter(g):
            b = g % _NBUF
            return pltpu.async_copy(
                bufs[b], out_hbm.at[pl.ds(base + g * chunk, chunk)], sem_s[b])

        def scale(b):
            def row_body(i, c2):
                for j in range(D_MODEL // 16):
                    sl = pl.ds(j * 16, 16)
                    bufs[b][i, sl] = bufs[b][i, sl] * SCALE
                return c2
            lax.fori_loop(0, chunk, row_body, 0, unroll=4)

        K = _NBUF - 1
        gather_h = [None] * n_chunks
        scatter_h = [None] * n_chunks
        for g in range(min(K, n_chunks)):
            gather_h[g] = start_gather(g)
        for g in range(n_chunks):
            ng = g + K
            if ng < n_chunks:
                if ng - _NBUF >= 0:
                    scatter_h[ng - _NBUF].wait()
                gather_h[ng] = start_gather(ng)
            gather_h[g].wait()
            scale(g % _NBUF)
            scatter_h[g] = start_scatter(g)
        for g in range(max(0, n_chunks - _NBUF), n_chunks):
            scatter_h[g].wait()

    return k(table, idx)


def kernel(x, table):
    total = x.shape[0] * x.shape[1]
    idx = x.reshape(total).astype(jnp.int32)
    out = _sc_embed(table, idx, total=total, chunk=512)
    return out.reshape(x.shape + (D_MODEL,))


# speedup vs baseline: 1.0694x; 1.0068x over previous
import functools
import math

import jax
import jax.numpy as jnp
from jax import lax
from jax.experimental import pallas as pl
from jax.experimental.pallas import tpu as pltpu
from jax.experimental.pallas import tpu_sc as plsc

D_MODEL = 64
SCALE = math.sqrt(D_MODEL)

_NC = 2
_NS = 16
_NW = _NC * _NS
_NBUF = 4


@functools.partial(jax.jit, static_argnames=("nrows", "seq"))
def _sc_embed(table, x, *, nrows, seq):
    rows_per_w = nrows // _NW
    mesh = plsc.VectorSubcoreMesh(core_axis_name="c", subcore_axis_name="s")

    @functools.partial(
        pl.kernel,
        mesh=mesh,
        out_type=jax.ShapeDtypeStruct((nrows, seq, D_MODEL), jnp.float32),
        scratch_types=[
            pltpu.VMEM((rows_per_w, seq), jnp.int32),
        ]
        + [pltpu.VMEM((seq, D_MODEL), jnp.float32) for _ in range(_NBUF)]
        + [pltpu.SemaphoreType.DMA for _ in range(2 * _NBUF)],
        compiler_params=pltpu.CompilerParams(use_tc_tiling_on_sc=False),
    )
    def k(table_hbm, x_hbm, out_hbm, idx_v, *bufs_and_sems):
        bufs = bufs_and_sems[:_NBUF]
        sem_g = bufs_and_sems[_NBUF:2 * _NBUF]
        sem_s = bufs_and_sems[2 * _NBUF:]

        wid = lax.axis_index("s") * _NC + lax.axis_index("c")
        row0 = wid * rows_per_w
        pltpu.sync_copy(x_hbm.at[pl.ds(row0, rows_per_w), :], idx_v)

        def start_gather(g, b):
            return pltpu.async_copy(table_hbm.at[idx_v.at[g]], bufs[b], sem_g[b])

        def wait_gather(g, b):
            pltpu.make_async_copy(
                table_hbm.at[idx_v.at[g]], bufs[b], sem_g[b]).wait()

        def start_scatter(g, b):
            return pltpu.async_copy(bufs[b], out_hbm.at[row0 + g], sem_s[b])

        def wait_scatter(b):
            pltpu.make_async_copy(bufs[b], out_hbm.at[row0], sem_s[b]).wait()

        def scale(b):
            def row_body(i, c2):
                for j in range(D_MODEL // 16):
                    sl = pl.ds(j * 16, 16)
                    bufs[b][i, sl] = bufs[b][i, sl] * SCALE
                return c2
            lax.fori_loop(0, seq, row_body, 0, unroll=4)

        n = rows_per_w
        start_gather(0, 0)
        start_gather(1, 1)
        start_gather(2, 2)
        wait_gather(0, 0)
        scale(0)
        start_scatter(0, 0)
        start_gather(3, 3)
        wait_gather(1, 1)
        scale(1)
        start_scatter(1, 1)

        def steady(p, carry):
            for b in range(_NBUF):
                g = 2 + p * _NBUF + b
                bb = (2 + b) % _NBUF
                bn = b % _NBUF
                wait_scatter(bn)
                start_gather(g + 2, bn)
                wait_gather(g, bb)
                scale(bb)
                start_scatter(g, bb)
            return carry

        lax.fori_loop(0, (n - 4) // _NBUF, steady, 0)

        wait_gather(n - 2, (n - 2) % _NBUF)
        scale((n - 2) % _NBUF)
        start_scatter(n - 2, (n - 2) % _NBUF)
        wait_gather(n - 1, (n - 1) % _NBUF)
        scale((n - 1) % _NBUF)
        start_scatter(n - 1, (n - 1) % _NBUF)
        for b in range(_NBUF):
            wait_scatter(b)

    return k(table, x)


def kernel(x, table):
    if x.dtype != jnp.int32:
        x = x.astype(jnp.int32)
    return _sc_embed(table, x, nrows=x.shape[0], seq=x.shape[1])
